# Initial kernel scaffold; baseline (speedup 1.0000x reference)
import jax
import jax.numpy as jnp
from jax.experimental import pallas as pl


def kernel(x, edge_index, edge_attr, W1, b1, W2, b2, W_ih, b_ih, W_hh, b_hh):
    raise NotImplementedError("write your pallas kernel here")


# speedup vs baseline: 2.9093x; 2.9093x over previous
import functools

import jax
import jax.numpy as jnp
from jax import lax
from jax.experimental import pallas as pl
from jax.experimental.pallas import tpu as pltpu
from jax.experimental.pallas import tpu_sc as plsc

NC = 2
NS = 16
NW = NC * NS
C = 80
AGG_W = 80


def _proj_body(x_ref, wa_ref, wb_ref, xa_ref, xb_ref):
    xr = x_ref[...]
    xa_ref[...] = jnp.dot(xr, wa_ref[...], preferred_element_type=jnp.float32)
    xb_ref[...] = jnp.dot(xr, wb_ref[...], preferred_element_type=jnp.float32)


def _eterm_body(ea_ref, wc_ref, b1_ref, out_ref):
    out_ref[...] = (
        jnp.dot(ea_ref[...], wc_ref[...], preferred_element_type=jnp.float32)
        + b1_ref[...]
    )


def _sc_edge_body(nblk, n_per_sub, xa_hbm, xb_hbm, et_hbm, src_hbm, dst_hbm,
                  zeros_hbm, tmpl_hbm, out_hbm, src_v, dst_v, ga, gb, et_v,
                  scat, haggr, sem_a, sem_b):
    cid = lax.axis_index("c")
    sid = lax.axis_index("s")
    wid = cid * NS + sid

    row0 = sid * n_per_sub
    pltpu.sync_copy(zeros_hbm.at[pl.ds(row0, n_per_sub)],
                    haggr.at[pl.ds(row0, n_per_sub)])

    pltpu.sync_copy(src_hbm.at[wid], src_v)
    pltpu.sync_copy(dst_hbm.at[wid], dst_v)

    pltpu.sync_copy(tmpl_hbm, scat)

    plsc.subcore_barrier()

    def _chunk(c, carry):
        cp_a = pltpu.async_copy(xa_hbm.at[src_v.at[c]], ga, sem_a)
        cp_b = pltpu.async_copy(xb_hbm.at[dst_v.at[c]], gb, sem_b)
        pltpu.sync_copy(et_hbm.at[wid, c], et_v)
        cp_a.wait()
        cp_b.wait()

        def _edge(e, ecarry):
            for v in range(4):
                sl = pl.ds(v * 16, 16)
                t = ga[e, sl] + gb[e, sl] + et_v[e, sl]
                scat[e, sl] = jnp.maximum(t, 0.0)
            return ecarry

        lax.fori_loop(0, C, _edge, 0, unroll=2)
        pltpu.sync_copy(scat, haggr.at[dst_v.at[c]], add=True)
        return carry

    lax.fori_loop(0, nblk, _chunk, 0)

    plsc.subcore_barrier()
    pltpu.sync_copy(haggr.at[pl.ds(row0, n_per_sub)],
                    out_hbm.at[cid, pl.ds(row0, n_per_sub)])


def _gru_body(hp_ref, x_ref, w2e_ref, wih_ref, bih_ref, whh_ref, bhh_ref, out_ref):
    d = x_ref.shape[1]
    hsum = hp_ref[0] + hp_ref[1]
    aggr = jnp.dot(hsum, w2e_ref[...], preferred_element_type=jnp.float32)
    gi = jnp.dot(aggr, wih_ref[...], preferred_element_type=jnp.float32) + bih_ref[...]
    xr = x_ref[...]
    gh = jnp.dot(xr, whh_ref[...], preferred_element_type=jnp.float32) + bhh_ref[...]
    r = jax.nn.sigmoid(gi[:, 0:d] + gh[:, 0:d])
    z = jax.nn.sigmoid(gi[:, d:2 * d] + gh[:, d:2 * d])
    n = jnp.tanh(gi[:, 2 * d:3 * d] + r * gh[:, 2 * d:3 * d])
    out_ref[...] = (1.0 - z) * n + z * xr


def kernel(x, edge_index, edge_attr, W1, b1, W2, b2, W_ih, b_ih, W_hh, b_hh):
    N, D = x.shape
    E = edge_index.shape[1]
    H = W1.shape[0]
    DE = edge_attr.shape[1]
    assert E % (NW * C) == 0
    nblk = E // (NW * C)
    n_per_sub = -(-N // (NS * 8)) * 8
    npad = n_per_sub * NS

    WaT = W1[:, :D].T
    WbT = W1[:, D:2 * D].T
    WcT = W1[:, 2 * D:].T

    rows = 1000
    xa, xb = pl.pallas_call(
        _proj_body,
        grid=(N // rows,),
        in_specs=[
            pl.BlockSpec((rows, D), lambda i: (i, 0)),
            pl.BlockSpec((D, H), lambda i: (0, 0)),
            pl.BlockSpec((D, H), lambda i: (0, 0)),
        ],
        out_specs=[
            pl.BlockSpec((rows, H), lambda i: (i, 0)),
            pl.BlockSpec((rows, H), lambda i: (i, 0)),
        ],
        out_shape=[
            jax.ShapeDtypeStruct((N, H), jnp.float32),
            jax.ShapeDtypeStruct((N, H), jnp.float32),
        ],
    )(x, WaT, WbT)

    erows = 2000
    et = pl.pallas_call(
        _eterm_body,
        grid=(E // erows,),
        in_specs=[
            pl.BlockSpec((erows, DE), lambda i: (i, 0)),
            pl.BlockSpec((DE, H), lambda i: (0, 0)),
            pl.BlockSpec((1, H), lambda i: (0, 0)),
        ],
        out_specs=pl.BlockSpec((erows, H), lambda i: (i, 0)),
        out_shape=jax.ShapeDtypeStruct((E, H), jnp.float32),
    )(edge_attr, WcT, b1.reshape(1, H))

    src_r = edge_index[0].reshape(NW, nblk, C)
    dst_r = edge_index[1].reshape(NW, nblk, C)
    et_r = et.reshape(NW, nblk, C, H)
    zeros = jnp.zeros((npad, AGG_W), jnp.float32)
    tmpl = jnp.zeros((C, AGG_W), jnp.float32).at[:, H].set(1.0)

    mesh = plsc.VectorSubcoreMesh(core_axis_name="c", subcore_axis_name="s",
                                  num_cores=NC, num_subcores=NS)
    hpart = pl.kernel(
        functools.partial(_sc_edge_body, nblk, n_per_sub),
        out_type=jax.ShapeDtypeStruct((NC, npad, AGG_W), jnp.float32),
        mesh=mesh,
        scratch_types=[
            pltpu.VMEM((nblk, C), jnp.int32),
            pltpu.VMEM((nblk, C), jnp.int32),
            pltpu.VMEM((C, H), jnp.float32),
            pltpu.VMEM((C, H), jnp.float32),
            pltpu.VMEM((C, H), jnp.float32),
            pltpu.VMEM((C, AGG_W), jnp.float32),
            pltpu.VMEM_SHARED((npad, AGG_W), jnp.float32),
            pltpu.SemaphoreType.DMA,
            pltpu.SemaphoreType.DMA,
        ],
        compiler_params=pltpu.CompilerParams(use_tc_tiling_on_sc=False),
    )(xa, xb, et_r, src_r, dst_r, zeros, tmpl)

    W2e = jnp.concatenate(
        [W2.T, b2.reshape(1, D), jnp.zeros((AGG_W - H - 1, D), jnp.float32)],
        axis=0)
    out = pl.pallas_call(
        _gru_body,
        grid=(N // rows,),
        in_specs=[
            pl.BlockSpec((NC, rows, AGG_W), lambda i: (0, i, 0)),
            pl.BlockSpec((rows, D), lambda i: (i, 0)),
            pl.BlockSpec((AGG_W, D), lambda i: (0, 0)),
            pl.BlockSpec((D, 3 * D), lambda i: (0, 0)),
            pl.BlockSpec((1, 3 * D), lambda i: (0, 0)),
            pl.BlockSpec((D, 3 * D), lambda i: (0, 0)),
            pl.BlockSpec((1, 3 * D), lambda i: (0, 0)),
        ],
        out_specs=pl.BlockSpec((rows, D), lambda i: (i, 0)),
        out_shape=jax.ShapeDtypeStruct((N, D), jnp.float32),
    )(hpart, x, W2e, W_ih.T, b_ih.reshape(1, 3 * D), W_hh.T,
      b_hh.reshape(1, 3 * D))
    return out


# speedup vs baseline: 3.3975x; 1.1678x over previous
import functools

import jax
import jax.numpy as jnp
from jax import lax
from jax.experimental import pallas as pl
from jax.experimental.pallas import tpu as pltpu
from jax.experimental.pallas import tpu_sc as plsc

NC = 2
NS = 16
NW = NC * NS
C = 80
AGG_W = 72


def _proj_body(x_ref, w_ref, xab_ref):
    xab_ref[...] = jnp.dot(x_ref[...], w_ref[...],
                           preferred_element_type=jnp.float32)


def _eterm_body(ea_ref, wc_ref, b1_ref, out_ref):
    out_ref[...] = (
        jnp.dot(ea_ref[...], wc_ref[...], preferred_element_type=jnp.float32)
        + b1_ref[...]
    )


def _sc_edge_body(nblk, n_per_sub, xab_hbm, et_hbm, src_hbm, dst_hbm,
                  zeros_hbm, tmpl_hbm, out_hbm, src_v, dst_v,
                  ga0, ga1, gb0, gb1, et0, et1, sc0, sc1, haggr,
                  sa0, sa1, sb0, sb1, se0, se1):
    cid = lax.axis_index("c")
    sid = lax.axis_index("s")
    wid = cid * NS + sid
    ga = (ga0, ga1)
    gb = (gb0, gb1)
    etv = (et0, et1)
    scat = (sc0, sc1)
    sa = (sa0, sa1)
    sb = (sb0, sb1)
    se = (se0, se1)
    ebase = wid * (nblk * C * 64)

    row0 = sid * n_per_sub
    zrows = zeros_hbm.shape[0]

    def _zero(k, carry):
        pltpu.sync_copy(zeros_hbm,
                        haggr.at[pl.ds(row0 + k * zrows, zrows)])
        return carry

    lax.fori_loop(0, n_per_sub // zrows, _zero, 0)

    pltpu.sync_copy(src_hbm.at[wid], src_v)
    pltpu.sync_copy(dst_hbm.at[wid], dst_v)

    pltpu.sync_copy(tmpl_hbm, scat[0])
    pltpu.sync_copy(tmpl_hbm, scat[1])

    plsc.subcore_barrier()

    def _start(c, b):
        pltpu.async_copy(xab_hbm.at[src_v.at[c]], ga[b], sa[b])
        pltpu.async_copy(xab_hbm.at[dst_v.at[c]], gb[b], sb[b])
        pltpu.async_copy(et_hbm.at[pl.ds(ebase + c * (C * 64), C * 64)],
                         etv[b], se[b])

    def _step(c, b):
        pltpu.make_async_copy(xab_hbm.at[src_v.at[c]], ga[b], sa[b]).wait()
        pltpu.make_async_copy(xab_hbm.at[dst_v.at[c]], gb[b], sb[b]).wait()
        pltpu.make_async_copy(
            et_hbm.at[pl.ds(ebase + c * (C * 64), C * 64)], etv[b],
            se[b]).wait()

        def _pair(e2, carry):
            for p in range(2):
                e = e2 * 2 + p
                for v in range(4):
                    t = (ga[b][e, pl.ds(v * 16, 16)]
                         + gb[b][e, pl.ds(64 + v * 16, 16)]
                         + etv[b][pl.ds(e2 * 128 + p * 64 + v * 16, 16)])
                    scat[b][e, pl.ds(v * 16, 16)] = jnp.maximum(t, 0.0)
            return carry

        lax.fori_loop(0, C // 2, _pair, 0, unroll=2)
        pltpu.sync_copy(scat[b], haggr.at[dst_v.at[c]], add=True)

    _start(0, 0)

    def _outer(i, carry):
        c0 = i * 2
        for b in range(2):
            c = c0 + b
            _start(c + 1, 1 - b)
            _step(c, b)
        return carry

    lax.fori_loop(0, (nblk - 1) // 2, _outer, 0)
    _step(nblk - 1, (nblk - 1) % 2)

    plsc.subcore_barrier()
    pltpu.sync_copy(haggr.at[pl.ds(row0, n_per_sub)],
                    out_hbm.at[cid, pl.ds(row0, n_per_sub)])


def _gru_body(hp_ref, x_ref, w2e_ref, wih_ref, bih_ref, whh_ref, bhh_ref, out_ref):
    d = x_ref.shape[1]
    hsum = hp_ref[0] + hp_ref[1]
    aggr = jnp.dot(hsum, w2e_ref[...], preferred_element_type=jnp.float32)
    gi = jnp.dot(aggr, wih_ref[...], preferred_element_type=jnp.float32) + bih_ref[...]
    xr = x_ref[...]
    gh = jnp.dot(xr, whh_ref[...], preferred_element_type=jnp.float32) + bhh_ref[...]
    r = jax.nn.sigmoid(gi[:, 0:d] + gh[:, 0:d])
    z = jax.nn.sigmoid(gi[:, d:2 * d] + gh[:, d:2 * d])
    n = jnp.tanh(gi[:, 2 * d:3 * d] + r * gh[:, 2 * d:3 * d])
    out_ref[...] = (1.0 - z) * n + z * xr


def kernel(x, edge_index, edge_attr, W1, b1, W2, b2, W_ih, b_ih, W_hh, b_hh):
    N, D = x.shape
    E = edge_index.shape[1]
    H = W1.shape[0]
    DE = edge_attr.shape[1]
    assert E % (NW * C) == 0 and C % 16 == 0
    nblk = E // (NW * C)
    n_per_sub = -(-N // (NS * 8)) * 8
    npad = n_per_sub * NS

    W_comb = jnp.concatenate([W1[:, :D].T, W1[:, D:2 * D].T], axis=1)
    rows = 1000
    xab = pl.pallas_call(
        _proj_body,
        grid=(N // rows,),
        in_specs=[
            pl.BlockSpec((rows, D), lambda i: (i, 0)),
            pl.BlockSpec((D, 2 * H), lambda i: (0, 0)),
        ],
        out_specs=pl.BlockSpec((rows, 2 * H), lambda i: (i, 0)),
        out_shape=jax.ShapeDtypeStruct((N, 2 * H), jnp.float32),
    )(x, W_comb)

    WcT = W1[:, 2 * D:].T
    Wbig = jnp.zeros((2 * DE, 2 * H), jnp.float32)
    Wbig = Wbig.at[0:DE, 0:H].set(WcT).at[DE:2 * DE, H:2 * H].set(WcT)
    b1b = jnp.concatenate([b1, b1]).reshape(1, 2 * H)
    ea2 = edge_attr.reshape(E // 2, 2 * DE)
    erows = 2000
    et = pl.pallas_call(
        _eterm_body,
        grid=(E // 2 // erows,),
        in_specs=[
            pl.BlockSpec((erows, 2 * DE), lambda i: (i, 0)),
            pl.BlockSpec((2 * DE, 2 * H), lambda i: (0, 0)),
            pl.BlockSpec((1, 2 * H), lambda i: (0, 0)),
        ],
        out_specs=pl.BlockSpec((erows, 2 * H), lambda i: (i, 0)),
        out_shape=jax.ShapeDtypeStruct((E // 2, 2 * H), jnp.float32),
    )(ea2, Wbig, b1b)
    et_flat = et.reshape(E * H)

    src_r = edge_index[0].reshape(NW, nblk, C)
    dst_r = edge_index[1].reshape(NW, nblk, C)
    assert n_per_sub % 8 == 0
    zeros = jnp.zeros((8, AGG_W), jnp.float32)
    tmpl = jnp.zeros((C, AGG_W), jnp.float32).at[:, H].set(1.0)

    mesh = plsc.VectorSubcoreMesh(core_axis_name="c", subcore_axis_name="s",
                                  num_cores=NC, num_subcores=NS)
    hpart = pl.kernel(
        functools.partial(_sc_edge_body, nblk, n_per_sub),
        out_type=jax.ShapeDtypeStruct((NC, npad, AGG_W), jnp.float32),
        mesh=mesh,
        scratch_types=[
            pltpu.VMEM((nblk, C), jnp.int32),
            pltpu.VMEM((nblk, C), jnp.int32),
            pltpu.VMEM((C, 2 * H), jnp.float32),
            pltpu.VMEM((C, 2 * H), jnp.float32),
            pltpu.VMEM((C, 2 * H), jnp.float32),
            pltpu.VMEM((C, 2 * H), jnp.float32),
            pltpu.VMEM((C * 64,), jnp.float32),
            pltpu.VMEM((C * 64,), jnp.float32),
            pltpu.VMEM((C, AGG_W), jnp.float32),
            pltpu.VMEM((C, AGG_W), jnp.float32),
            pltpu.VMEM_SHARED((npad, AGG_W), jnp.float32),
            pltpu.SemaphoreType.DMA,
            pltpu.SemaphoreType.DMA,
            pltpu.SemaphoreType.DMA,
            pltpu.SemaphoreType.DMA,
            pltpu.SemaphoreType.DMA,
            pltpu.SemaphoreType.DMA,
        ],
        compiler_params=pltpu.CompilerParams(use_tc_tiling_on_sc=False),
    )(xab, et_flat, src_r, dst_r, zeros, tmpl)

    W2e = jnp.concatenate(
        [W2.T, b2.reshape(1, D), jnp.zeros((AGG_W - H - 1, D), jnp.float32)],
        axis=0)
    out = pl.pallas_call(
        _gru_body,
        grid=(N // rows,),
        in_specs=[
            pl.BlockSpec((NC, rows, AGG_W), lambda i: (0, i, 0)),
            pl.BlockSpec((rows, D), lambda i: (i, 0)),
            pl.BlockSpec((AGG_W, D), lambda i: (0, 0)),
            pl.BlockSpec((D, 3 * D), lambda i: (0, 0)),
            pl.BlockSpec((1, 3 * D), lambda i: (0, 0)),
            pl.BlockSpec((D, 3 * D), lambda i: (0, 0)),
            pl.BlockSpec((1, 3 * D), lambda i: (0, 0)),
        ],
        out_specs=pl.BlockSpec((rows, D), lambda i: (i, 0)),
        out_shape=jax.ShapeDtypeStruct((N, D), jnp.float32),
    )(hpart, x, W2e, W_ih.T, b_ih.reshape(1, 3 * D), W_hh.T,
      b_hh.reshape(1, 3 * D))
    return out


# speedup vs baseline: 4.0691x; 1.1977x over previous
import functools

import jax
import jax.numpy as jnp
from jax import lax
from jax.experimental import pallas as pl
from jax.experimental.pallas import tpu as pltpu
from jax.experimental.pallas import tpu_sc as plsc

NC = 2
NS = 16
NW = NC * NS
C = 80
AGG_W = 72


def _proj_body(x_ref, wa_ref, wb_ref, xa_ref, xb_ref):
    xr = x_ref[...]
    xa_ref[...] = jnp.dot(xr, wa_ref[...], preferred_element_type=jnp.float32)
    xb_ref[...] = jnp.dot(xr, wb_ref[...], preferred_element_type=jnp.float32)


def _eterm_body(ea_lo_ref, ea_hi_ref, wc_ref, b1_ref, out_ref):
    lo = jnp.dot(ea_lo_ref[...], wc_ref[...], preferred_element_type=jnp.float32)
    hi = jnp.dot(ea_hi_ref[...], wc_ref[...], preferred_element_type=jnp.float32)
    out_ref[...] = jnp.concatenate([lo, hi], axis=1) + b1_ref[...]


def _sc_edge_body(nblk, n_per_sub, xa_hbm, xb_hbm, et_hbm, src_hbm, dst_hbm,
                  zeros_hbm, tmpl_hbm, out_hbm, src_v, dst_v,
                  ga0, ga1, ga2, gb0, gb1, gb2, et0, et1, et2, sc0, sc1,
                  haggr, sa0, sa1, sa2, sb0, sb1, sb2, se0, se1, se2,
                  ss0, ss1):
    cid = lax.axis_index("c")
    sid = lax.axis_index("s")
    wid = cid * NS + sid
    ga = (ga0, ga1, ga2)
    gb = (gb0, gb1, gb2)
    etv = (et0, et1, et2)
    scat = (sc0, sc1)
    sa = (sa0, sa1, sa2)
    sb = (sb0, sb1, sb2)
    se = (se0, se1, se2)
    ss = (ss0, ss1)
    erow0 = sid * (nblk * C)
    ecol0 = cid * 64

    row0 = sid * n_per_sub
    zrows = zeros_hbm.shape[0]

    def _zero(k, carry):
        pltpu.sync_copy(zeros_hbm,
                        haggr.at[pl.ds(row0 + k * zrows, zrows)])
        return carry

    lax.fori_loop(0, n_per_sub // zrows, _zero, 0)

    pltpu.sync_copy(src_hbm.at[wid], src_v)
    pltpu.sync_copy(dst_hbm.at[wid], dst_v)

    pltpu.sync_copy(tmpl_hbm, scat[0])
    pltpu.sync_copy(tmpl_hbm, scat[1])

    plsc.subcore_barrier()

    def _startg(c, p):
        pltpu.async_copy(xa_hbm.at[src_v.at[c]], ga[p], sa[p])
        pltpu.async_copy(xb_hbm.at[dst_v.at[c]], gb[p], sb[p])
        pltpu.async_copy(
            et_hbm.at[pl.ds(erow0 + c * C, C), pl.ds(ecol0, 64)],
            etv[p], se[p])

    def _waitg(c, p):
        pltpu.make_async_copy(xa_hbm.at[src_v.at[c]], ga[p], sa[p]).wait()
        pltpu.make_async_copy(xb_hbm.at[dst_v.at[c]], gb[p], sb[p]).wait()
        pltpu.make_async_copy(
            et_hbm.at[pl.ds(erow0 + c * C, C), pl.ds(ecol0, 64)],
            etv[p], se[p]).wait()

    def _compute(c, p, q):
        def _edge(e, carry):
            for v in range(4):
                t = (ga[p][e, pl.ds(v * 16, 16)]
                     + gb[p][e, pl.ds(v * 16, 16)]
                     + etv[p][e, pl.ds(v * 16, 16)])
                scat[q][e, pl.ds(v * 16, 16)] = jnp.maximum(t, 0.0)
            return carry

        lax.fori_loop(0, C, _edge, 0, unroll=4)

    def _fires(c, q):
        pltpu.async_copy(scat[q], haggr.at[dst_v.at[c]], ss[q], add=True)

    def _waits(c, q):
        pltpu.make_async_copy(scat[q], haggr.at[dst_v.at[c]], ss[q]).wait()

    _startg(0, 0)
    _startg(1, 1)

    _waitg(0, 0)
    _startg(2, 2)
    _compute(0, 0, 0)
    _fires(0, 0)

    _waitg(1, 1)
    _startg(3, 0)
    _compute(1, 1, 1)
    _fires(1, 1)

    def _outer(i, carry):
        c0 = 2 + i * 6
        for b in range(6):
            c = c0 + b
            gp = (2 + b) % 3
            sp = b % 2
            _waits(c - 2, sp)
            _waitg(c, gp)
            _startg(c + 2, (4 + b) % 3)
            _compute(c, gp, sp)
            _fires(c, sp)
        return carry

    assert (nblk - 5) % 6 == 0
    lax.fori_loop(0, (nblk - 5) // 6, _outer, 0)

    for c in (nblk - 3, nblk - 2, nblk - 1):
        gp = c % 3
        sp = c % 2
        _waits(c - 2, sp)
        _waitg(c, gp)
        if c + 2 < nblk:
            _startg(c + 2, (c + 2) % 3)
        _compute(c, gp, sp)
        _fires(c, sp)
    _waits(nblk - 2, (nblk - 2) % 2)
    _waits(nblk - 1, (nblk - 1) % 2)

    plsc.subcore_barrier()
    pltpu.sync_copy(haggr.at[pl.ds(row0, n_per_sub)],
                    out_hbm.at[cid, pl.ds(row0, n_per_sub)])


def _gru_body(hp_ref, x_ref, w2e_ref, wih_ref, bih_ref, whh_ref, bhh_ref, out_ref):
    d = x_ref.shape[1]
    hsum = hp_ref[0] + hp_ref[1]
    aggr = jnp.dot(hsum, w2e_ref[...], preferred_element_type=jnp.float32)
    gi = jnp.dot(aggr, wih_ref[...], preferred_element_type=jnp.float32) + bih_ref[...]
    xr = x_ref[...]
    gh = jnp.dot(xr, whh_ref[...], preferred_element_type=jnp.float32) + bhh_ref[...]
    r = jax.nn.sigmoid(gi[:, 0:d] + gh[:, 0:d])
    z = jax.nn.sigmoid(gi[:, d:2 * d] + gh[:, d:2 * d])
    n = jnp.tanh(gi[:, 2 * d:3 * d] + r * gh[:, 2 * d:3 * d])
    out_ref[...] = (1.0 - z) * n + z * xr


def kernel(x, edge_index, edge_attr, W1, b1, W2, b2, W_ih, b_ih, W_hh, b_hh):
    N, D = x.shape
    E = edge_index.shape[1]
    H = W1.shape[0]
    DE = edge_attr.shape[1]
    assert E % (NW * C) == 0 and C % 16 == 0
    nblk = E // (NW * C)
    n_per_sub = -(-N // (NS * 8)) * 8
    npad = n_per_sub * NS

    rows = 1000
    xa, xb = pl.pallas_call(
        _proj_body,
        grid=(N // rows,),
        in_specs=[
            pl.BlockSpec((rows, D), lambda i: (i, 0)),
            pl.BlockSpec((D, H), lambda i: (0, 0)),
            pl.BlockSpec((D, H), lambda i: (0, 0)),
        ],
        out_specs=[
            pl.BlockSpec((rows, H), lambda i: (i, 0)),
            pl.BlockSpec((rows, H), lambda i: (i, 0)),
        ],
        out_shape=[
            jax.ShapeDtypeStruct((N, H), jnp.float32),
            jax.ShapeDtypeStruct((N, H), jnp.float32),
        ],
    )(x, W1[:, :D].T, W1[:, D:2 * D].T)

    WcT = W1[:, 2 * D:].T
    b1b = jnp.concatenate([b1, b1]).reshape(1, 2 * H)
    erows = 2000
    nhalf = E // 2 // erows
    et = pl.pallas_call(
        _eterm_body,
        grid=(nhalf,),
        in_specs=[
            pl.BlockSpec((erows, DE), lambda i: (i, 0)),
            pl.BlockSpec((erows, DE), lambda i, _n=nhalf: (i + _n, 0)),
            pl.BlockSpec((DE, H), lambda i: (0, 0)),
            pl.BlockSpec((1, 2 * H), lambda i: (0, 0)),
        ],
        out_specs=pl.BlockSpec((erows, 2 * H), lambda i: (i, 0)),
        out_shape=jax.ShapeDtypeStruct((E // 2, 2 * H), jnp.float32),
    )(edge_attr, edge_attr, WcT, b1b)

    src_r = edge_index[0].reshape(NW, nblk, C)
    dst_r = edge_index[1].reshape(NW, nblk, C)
    assert n_per_sub % 8 == 0
    zeros = jnp.zeros((8, AGG_W), jnp.float32)
    tmpl = jnp.zeros((C, AGG_W), jnp.float32).at[:, H].set(1.0)

    mesh = plsc.VectorSubcoreMesh(core_axis_name="c", subcore_axis_name="s",
                                  num_cores=NC, num_subcores=NS)
    hpart = pl.kernel(
        functools.partial(_sc_edge_body, nblk, n_per_sub),
        out_type=jax.ShapeDtypeStruct((NC, npad, AGG_W), jnp.float32),
        mesh=mesh,
        scratch_types=[
            pltpu.VMEM((nblk, C), jnp.int32),
            pltpu.VMEM((nblk, C), jnp.int32),
            pltpu.VMEM((C, H), jnp.float32),
            pltpu.VMEM((C, H), jnp.float32),
            pltpu.VMEM((C, H), jnp.float32),
            pltpu.VMEM((C, H), jnp.float32),
            pltpu.VMEM((C, H), jnp.float32),
            pltpu.VMEM((C, H), jnp.float32),
            pltpu.VMEM((C, H), jnp.float32),
            pltpu.VMEM((C, H), jnp.float32),
            pltpu.VMEM((C, H), jnp.float32),
            pltpu.VMEM((C, AGG_W), jnp.float32),
            pltpu.VMEM((C, AGG_W), jnp.float32),
            pltpu.VMEM_SHARED((npad, AGG_W), jnp.float32),
            pltpu.SemaphoreType.DMA,
            pltpu.SemaphoreType.DMA,
            pltpu.SemaphoreType.DMA,
            pltpu.SemaphoreType.DMA,
            pltpu.SemaphoreType.DMA,
            pltpu.SemaphoreType.DMA,
            pltpu.SemaphoreType.DMA,
            pltpu.SemaphoreType.DMA,
            pltpu.SemaphoreType.DMA,
            pltpu.SemaphoreType.DMA,
            pltpu.SemaphoreType.DMA,
        ],
        compiler_params=pltpu.CompilerParams(use_tc_tiling_on_sc=False),
    )(xa, xb, et, src_r, dst_r, zeros, tmpl)

    W2e = jnp.concatenate(
        [W2.T, b2.reshape(1, D), jnp.zeros((AGG_W - H - 1, D), jnp.float32)],
        axis=0)
    out = pl.pallas_call(
        _gru_body,
        grid=(N // rows,),
        in_specs=[
            pl.BlockSpec((NC, rows, AGG_W), lambda i: (0, i, 0)),
            pl.BlockSpec((rows, D), lambda i: (i, 0)),
            pl.BlockSpec((AGG_W, D), lambda i: (0, 0)),
            pl.BlockSpec((D, 3 * D), lambda i: (0, 0)),
            pl.BlockSpec((1, 3 * D), lambda i: (0, 0)),
            pl.BlockSpec((D, 3 * D), lambda i: (0, 0)),
            pl.BlockSpec((1, 3 * D), lambda i: (0, 0)),
        ],
        out_specs=pl.BlockSpec((rows, D), lambda i: (i, 0)),
        out_shape=jax.ShapeDtypeStruct((N, D), jnp.float32),
    )(hpart, x, W2e, W_ih.T, b_ih.reshape(1, 3 * D), W_hh.T,
      b_hh.reshape(1, 3 * D))
    return out
